# baseline (speedup 1.0000x reference)
import functools

import jax
import jax.numpy as jnp
from jax import lax
from jax.experimental import pallas as pl
from jax.experimental.pallas import tpu as pltpu
from jax.experimental.pallas import tpu_sc as plsc

N = 10000
E = 320000
D = 128
NF = 1024
EA = 16

NC = 2
NS = 16
NW = NC * NS
CH = 128
NCHUNK = E // CH
NP = 10240
RP = NP // NS
K = 2
ES = E // K
NCS = NCHUNK // K

_MESH = plsc.VectorSubcoreMesh(core_axis_name="c", subcore_axis_name="s",
                               num_cores=NC, num_subcores=NS)


@functools.cache
def _make_gather(nchunk):
    e_sl = nchunk * CH

    @functools.partial(
        pl.kernel,
        out_type=(jax.ShapeDtypeStruct((e_sl, D), jnp.float32),
                  jax.ShapeDtypeStruct((e_sl, D), jnp.float32)),
        mesh=_MESH,
        scratch_types=[
            pltpu.VMEM((2, 1, CH), jnp.int32),
            pltpu.VMEM((2, 1, CH), jnp.int32),
            pltpu.VMEM((2, CH, D), jnp.float32),
            pltpu.VMEM((2, CH, D), jnp.float32),
        ] + [pltpu.SemaphoreType.DMA] * 12,
    )
    def gather(x_hbm, rows_hbm, cols_hbm, xs_hbm, xt_hbm,
               idxr_v, idxc_v, xs_v, xt_v,
               sir0, sir1, sic0, sic1, sgr0, sgr1, sgc0, sgc1,
               swr0, swr1, swc0, swc1):
        wid = lax.axis_index("s") * NC + lax.axis_index("c")
        sir = (sir0, sir1); sic = (sic0, sic1)
        sgr = (sgr0, sgr1); sgc = (sgc0, sgc1)
        swr = (swr0, swr1); swc = (swc0, swc1)
        cnt_w = (nchunk - wid + NW - 1) // NW
        pairs_end = (cnt_w // 2) * 2

        @pl.loop(0, pairs_end, step=2)
        def _(kk):
            hi = {}
            for b in (0, 1):
                j = wid + (kk + b) * NW
                hi[("r", b)] = pltpu.async_copy(rows_hbm.at[j], idxr_v.at[b],
                                                sir[b])
                hi[("c", b)] = pltpu.async_copy(cols_hbm.at[j], idxc_v.at[b],
                                                sic[b])
            hg = {}
            for b in (0, 1):
                hi[("r", b)].wait()
                hg[("r", b)] = pltpu.async_copy(x_hbm.at[idxr_v.at[b].at[0]],
                                                xs_v.at[b], sgr[b])
                hi[("c", b)].wait()
                hg[("c", b)] = pltpu.async_copy(x_hbm.at[idxc_v.at[b].at[0]],
                                                xt_v.at[b], sgc[b])
            hw = {}
            for b in (0, 1):
                j = wid + (kk + b) * NW
                hg[("r", b)].wait()
                hw[("r", b)] = pltpu.async_copy(
                    xs_v.at[b], xs_hbm.at[pl.ds(j * CH, CH)], swr[b])
                hg[("c", b)].wait()
                hw[("c", b)] = pltpu.async_copy(
                    xt_v.at[b], xt_hbm.at[pl.ds(j * CH, CH)], swc[b])
            for b in (0, 1):
                hw[("r", b)].wait()
                hw[("c", b)].wait()

        @pl.loop(pairs_end, cnt_w)
        def _(i):
            j = wid + i * NW
            pltpu.sync_copy(rows_hbm.at[j], idxr_v.at[0])
            pltpu.sync_copy(cols_hbm.at[j], idxc_v.at[0])
            pltpu.sync_copy(x_hbm.at[idxr_v.at[0].at[0]], xs_v.at[0])
            pltpu.sync_copy(x_hbm.at[idxc_v.at[0].at[0]], xt_v.at[0])
            pltpu.sync_copy(xs_v.at[0], xs_hbm.at[pl.ds(j * CH, CH)])
            pltpu.sync_copy(xt_v.at[0], xt_hbm.at[pl.ds(j * CH, CH)])

    return gather


@functools.partial(
    pl.kernel,
    out_type=jax.ShapeDtypeStruct((NC * NP, D), jnp.float32),
    mesh=_MESH,
    scratch_types=[
        pltpu.VMEM_SHARED((NP, D), jnp.float32),
        pltpu.VMEM((2, 1, CH), jnp.int32),
        pltpu.VMEM((CH, D), jnp.float32),
        pltpu.VMEM((CH, D), jnp.float32),
    ] + [pltpu.SemaphoreType.DMA] * 4,
)
def _sc_counts(rows_hbm, z128_hbm, ones_hbm, c_hbm,
               acc_sh, idx_v, ones_v, stg_v,
               si0, si1, sa0, sa1):
    cid = lax.axis_index("c")
    sid = lax.axis_index("s")
    wid = sid * NC + cid
    base = sid * RP
    si = (si0, si1); sa = (sa0, sa1)
    cnt_w = (NCHUNK - wid + NW - 1) // NW
    pairs_end = (cnt_w // 2) * 2

    pltpu.sync_copy(z128_hbm, stg_v)
    pltpu.sync_copy(ones_hbm, ones_v)

    @pl.loop(0, RP // CH)
    def _(k):
        pltpu.sync_copy(stg_v, acc_sh.at[pl.ds(base + k * CH, CH)])

    plsc.subcore_barrier()

    @pl.loop(0, pairs_end, step=2)
    def _(kk):
        hi = {}
        for b in (0, 1):
            j = wid + (kk + b) * NW
            hi[b] = pltpu.async_copy(rows_hbm.at[j], idx_v.at[b], si[b])
        ha = {}
        for b in (0, 1):
            hi[b].wait()
            ha[b] = pltpu.async_copy(ones_v, acc_sh.at[idx_v.at[b].at[0]],
                                     sa[b], add=True)
        for b in (0, 1):
            ha[b].wait()

    @pl.loop(pairs_end, cnt_w)
    def _(i):
        j = wid + i * NW
        pltpu.sync_copy(rows_hbm.at[j], idx_v.at[0])
        pltpu.sync_copy(ones_v, acc_sh.at[idx_v.at[0].at[0]], add=True)

    plsc.subcore_barrier()

    @pl.loop(0, RP // CH)
    def _(k):
        pltpu.sync_copy(acc_sh.at[pl.ds(base + k * CH, CH)], stg_v)
        pltpu.sync_copy(stg_v, c_hbm.at[pl.ds(cid * NP + base + k * CH, CH)])


@functools.cache
def _make_scatter(nchunk):
    @functools.partial(
        pl.kernel,
        out_type=jax.ShapeDtypeStruct((NC * NP, D), jnp.float32),
        mesh=_MESH,
        scratch_types=[
            pltpu.VMEM_SHARED((NP, D), jnp.float32),
            pltpu.VMEM((2, 1, CH), jnp.int32),
            pltpu.VMEM((2, CH, D), jnp.float32),
        ] + [pltpu.SemaphoreType.DMA] * 6,
    )
    def scatter(m_hbm, rows_hbm, z128_hbm, p_hbm,
                agg_sh, idx_v, m_v,
                si0, si1, sm0, sm1, sa0, sa1):
        cid = lax.axis_index("c")
        sid = lax.axis_index("s")
        wid = sid * NC + cid
        base = sid * RP
        si = (si0, si1); sm = (sm0, sm1); sa = (sa0, sa1)
        cnt_w = (nchunk - wid + NW - 1) // NW
        pairs_end = (cnt_w // 2) * 2

        pltpu.sync_copy(z128_hbm, m_v.at[0])

        @pl.loop(0, RP // CH)
        def _(k):
            pltpu.sync_copy(m_v.at[0], agg_sh.at[pl.ds(base + k * CH, CH)])

        plsc.subcore_barrier()

        @pl.loop(0, pairs_end, step=2)
        def _(kk):
            hi = {}
            for b in (0, 1):
                j = wid + (kk + b) * NW
                hi[("i", b)] = pltpu.async_copy(rows_hbm.at[j], idx_v.at[b],
                                                si[b])
                hi[("m", b)] = pltpu.async_copy(m_hbm.at[pl.ds(j * CH, CH)],
                                                m_v.at[b], sm[b])
            ha = {}
            for b in (0, 1):
                hi[("i", b)].wait()
                hi[("m", b)].wait()
                ha[b] = pltpu.async_copy(m_v.at[b],
                                         agg_sh.at[idx_v.at[b].at[0]],
                                         sa[b], add=True)
            for b in (0, 1):
                ha[b].wait()

        @pl.loop(pairs_end, cnt_w)
        def _(i):
            j = wid + i * NW
            pltpu.sync_copy(rows_hbm.at[j], idx_v.at[0])
            pltpu.sync_copy(m_hbm.at[pl.ds(j * CH, CH)], m_v.at[0])
            pltpu.sync_copy(m_v.at[0], agg_sh.at[idx_v.at[0].at[0]], add=True)

        plsc.subcore_barrier()

        @pl.loop(0, RP // CH)
        def _(k):
            pltpu.sync_copy(agg_sh.at[pl.ds(base + k * CH, CH)], m_v.at[0])
            pltpu.sync_copy(m_v.at[0],
                            p_hbm.at[pl.ds(cid * NP + base + k * CH, CH)])

    return scatter


EB = 2000


def _tc_mlp_body(xs_ref, xt_ref, ea_ref, w1bt_ref, w1a_ref, b1_ref, w2_ref,
                 m_ref):
    d = xs_ref[...] - xt_ref[...]
    radial = jnp.sqrt(jnp.sum(d * d, axis=1, keepdims=True))
    z = jnp.dot(ea_ref[...].astype(jnp.bfloat16), w1bt_ref[...],
                preferred_element_type=jnp.float32)
    z = z + radial * w1a_ref[...] + b1_ref[...]
    h = jnp.maximum(z, 0.2 * z)
    e = jnp.sum(h * w2_ref[...], axis=1, keepdims=True)
    m_ref[...] = d * jnp.tanh(e)


def _tc_mlp(xs, xt, ea, w1bt, w1a, b1, w2):
    e_sl = xs.shape[0]
    return pl.pallas_call(
        _tc_mlp_body,
        grid=(e_sl // EB,),
        in_specs=[
            pl.BlockSpec((EB, D), lambda i: (i, 0)),
            pl.BlockSpec((EB, D), lambda i: (i, 0)),
            pl.BlockSpec((EB, EA), lambda i: (i, 0)),
            pl.BlockSpec((EA, NF), lambda i: (0, 0)),
            pl.BlockSpec((1, NF), lambda i: (0, 0)),
            pl.BlockSpec((1, NF), lambda i: (0, 0)),
            pl.BlockSpec((1, NF), lambda i: (0, 0)),
        ],
        out_specs=pl.BlockSpec((EB, D), lambda i: (i, 0)),
        out_shape=jax.ShapeDtypeStruct((e_sl, D), jnp.float32),
    )(xs, xt, ea, w1bt, w1a, b1, w2)


NB = 2000


def _tc_combine_body(x_ref, p_ref, c_ref, vel_ref, vn_ref,
                     wv1_ref, bv1_ref, wv2_ref, bv2_ref, o_ref):
    cnt = c_ref[0, :, 0:1] + c_ref[1, :, 0:1]
    agg = p_ref[0]
    for q in range(1, K * NC):
        agg = agg + p_ref[q]
    agg = agg / jnp.maximum(cnt, 1.0)
    zv = vn_ref[...] * wv1_ref[...] + bv1_ref[...]
    hv = jnp.maximum(zv, 0.2 * zv)
    v_scale = jnp.sum(hv * wv2_ref[...], axis=1, keepdims=True) + bv2_ref[...]
    o_ref[...] = x_ref[...] + agg + vel_ref[...] * v_scale


def _tc_combine(x, p, c, vel, vn, wv1, bv1, wv2, bv2):
    return pl.pallas_call(
        _tc_combine_body,
        grid=(N // NB,),
        in_specs=[
            pl.BlockSpec((NB, D), lambda i: (i, 0)),
            pl.BlockSpec((K * NC, NB, D), lambda i: (0, i, 0)),
            pl.BlockSpec((NC, NB, D), lambda i: (0, i, 0)),
            pl.BlockSpec((NB, D), lambda i: (i, 0)),
            pl.BlockSpec((NB, 1), lambda i: (i, 0)),
            pl.BlockSpec((1, NF), lambda i: (0, 0)),
            pl.BlockSpec((1, NF), lambda i: (0, 0)),
            pl.BlockSpec((1, NF), lambda i: (0, 0)),
            pl.BlockSpec((1, 1), lambda i: (0, 0)),
        ],
        out_specs=pl.BlockSpec((NB, D), lambda i: (i, 0)),
        out_shape=jax.ShapeDtypeStruct((N, D), jnp.float32),
    )(x, p, c, vel, vn, wv1, bv1, wv2, bv2)


def kernel(x, vel_norm, vel, edge_index, edge_attr, W1, b1, W2,
           Wv1, bv1, Wv2, bv2):
    row = edge_index[0].astype(jnp.int32)
    col = edge_index[1].astype(jnp.int32)
    rows3d = row.reshape(NCHUNK, 1, CH)
    cols3d = col.reshape(NCHUNK, 1, CH)

    w1bt = W1[:, 1:].T.astype(jnp.bfloat16)
    w1a = W1[:, 0].reshape(1, NF)
    b1r = b1.reshape(1, NF)
    w2r = W2.reshape(1, NF)

    z128 = jnp.zeros((CH, D), jnp.float32)
    ones128 = jnp.ones((CH, D), jnp.float32)
    c = _sc_counts(rows3d, z128, ones128)

    gather = _make_gather(NCS)
    scatter = _make_scatter(NCS)
    parts = []
    for k in range(K):
        r_sl = rows3d[k * NCS:(k + 1) * NCS]
        c_sl = cols3d[k * NCS:(k + 1) * NCS]
        ea_sl = edge_attr[k * ES:(k + 1) * ES]
        xs, xt = gather(x, r_sl, c_sl)
        m = _tc_mlp(xs, xt, ea_sl, w1bt, w1a, b1r, w2r)
        parts.append(scatter(m, r_sl, z128))

    p = jnp.concatenate(parts).reshape(K * NC, NP, D)
    c = c.reshape(NC, NP, D)

    wv1 = Wv1.reshape(1, NF)
    bv1r = bv1.reshape(1, NF)
    wv2 = Wv2.reshape(1, NF)
    bv2r = bv2.reshape(1, 1)
    x_out = _tc_combine(x, p, c, vel, vel_norm, wv1, bv1r, wv2, bv2r)
    return (x_out, edge_attr)


# speedup vs baseline: 1.0267x; 1.0267x over previous
import functools

import jax
import jax.numpy as jnp
from jax import lax
from jax.experimental import pallas as pl
from jax.experimental.pallas import tpu as pltpu
from jax.experimental.pallas import tpu_sc as plsc

N = 10000
E = 320000
D = 128
NF = 1024
EA = 16

NC = 2
NS = 16
NW = NC * NS
CH = 128
NCHUNK = E // CH
NP = 10240
RP = NP // NS
K = 2
ES = E // K
NCS = NCHUNK // K

def acc_at_idx(acc, idx_v, b):
    return acc.at[idx_v.at[b].at[0]]


_MESH = plsc.VectorSubcoreMesh(core_axis_name="c", subcore_axis_name="s",
                               num_cores=NC, num_subcores=NS)


@functools.cache
def _make_gather(nchunk):
    e_sl = nchunk * CH

    @functools.partial(
        pl.kernel,
        out_type=(jax.ShapeDtypeStruct((e_sl, D), jnp.float32),
                  jax.ShapeDtypeStruct((e_sl, D), jnp.float32)),
        mesh=_MESH,
        scratch_types=[
            pltpu.VMEM((2, 1, CH), jnp.int32),
            pltpu.VMEM((2, 1, CH), jnp.int32),
            pltpu.VMEM((2, CH, D), jnp.float32),
            pltpu.VMEM((2, CH, D), jnp.float32),
        ] + [pltpu.SemaphoreType.DMA] * 12,
    )
    def gather(x_hbm, rows_hbm, cols_hbm, xs_hbm, xt_hbm,
               idxr_v, idxc_v, xs_v, xt_v,
               sir0, sir1, sic0, sic1, sgr0, sgr1, sgc0, sgc1,
               swr0, swr1, swc0, swc1):
        wid = lax.axis_index("s") * NC + lax.axis_index("c")
        sir = (sir0, sir1); sic = (sic0, sic1)
        sgr = (sgr0, sgr1); sgc = (sgc0, sgc1)
        swr = (swr0, swr1); swc = (swc0, swc1)
        cnt_w = (nchunk - wid + NW - 1) // NW
        pairs_end = (cnt_w // 2) * 2

        @pl.loop(0, pairs_end, step=2)
        def _(kk):
            hi = {}
            for b in (0, 1):
                j = wid + (kk + b) * NW
                hi[("r", b)] = pltpu.async_copy(rows_hbm.at[j], idxr_v.at[b],
                                                sir[b])
                hi[("c", b)] = pltpu.async_copy(cols_hbm.at[j], idxc_v.at[b],
                                                sic[b])
            hg = {}
            for b in (0, 1):
                hi[("r", b)].wait()
                hg[("r", b)] = pltpu.async_copy(x_hbm.at[idxr_v.at[b].at[0]],
                                                xs_v.at[b], sgr[b])
                hi[("c", b)].wait()
                hg[("c", b)] = pltpu.async_copy(x_hbm.at[idxc_v.at[b].at[0]],
                                                xt_v.at[b], sgc[b])
            hw = {}
            for b in (0, 1):
                j = wid + (kk + b) * NW
                hg[("r", b)].wait()
                hw[("r", b)] = pltpu.async_copy(
                    xs_v.at[b], xs_hbm.at[pl.ds(j * CH, CH)], swr[b])
                hg[("c", b)].wait()
                hw[("c", b)] = pltpu.async_copy(
                    xt_v.at[b], xt_hbm.at[pl.ds(j * CH, CH)], swc[b])
            for b in (0, 1):
                hw[("r", b)].wait()
                hw[("c", b)].wait()

        @pl.loop(pairs_end, cnt_w)
        def _(i):
            j = wid + i * NW
            pltpu.sync_copy(rows_hbm.at[j], idxr_v.at[0])
            pltpu.sync_copy(cols_hbm.at[j], idxc_v.at[0])
            pltpu.sync_copy(x_hbm.at[idxr_v.at[0].at[0]], xs_v.at[0])
            pltpu.sync_copy(x_hbm.at[idxc_v.at[0].at[0]], xt_v.at[0])
            pltpu.sync_copy(xs_v.at[0], xs_hbm.at[pl.ds(j * CH, CH)])
            pltpu.sync_copy(xt_v.at[0], xt_hbm.at[pl.ds(j * CH, CH)])

    return gather


@functools.cache
def _make_scatter(nchunk, with_counts=False):
    outs = [jax.ShapeDtypeStruct((NC * NP, D), jnp.float32)]
    if with_counts:
        outs.append(jax.ShapeDtypeStruct((NC * NP, D), jnp.float32))

    def scatter(m_hbm, rows_all_hbm, rows_hbm, z128_hbm, ones_hbm,
                p_hbm, *rest):
        if with_counts:
            c_hbm, agg_sh, idx_v, m_v, si0, si1, sm0, sm1, sa0, sa1 = rest
        else:
            agg_sh, idx_v, m_v, si0, si1, sm0, sm1, sa0, sa1 = rest
        cid = lax.axis_index("c")
        sid = lax.axis_index("s")
        wid = sid * NC + cid
        base = sid * RP
        si = (si0, si1); sm = (sm0, sm1); sa = (sa0, sa1)
        cnt_w = (nchunk - wid + NW - 1) // NW
        pairs_end = (cnt_w // 2) * 2

        pltpu.sync_copy(z128_hbm, m_v.at[0])

        @pl.loop(0, RP // CH)
        def _(k):
            pltpu.sync_copy(m_v.at[0], agg_sh.at[pl.ds(base + k * CH, CH)])

        plsc.subcore_barrier()

        @pl.loop(0, pairs_end, step=2)
        def _(kk):
            hi = {}
            for b in (0, 1):
                j = wid + (kk + b) * NW
                hi[("i", b)] = pltpu.async_copy(rows_hbm.at[j], idx_v.at[b],
                                                si[b])
                hi[("m", b)] = pltpu.async_copy(m_hbm.at[pl.ds(j * CH, CH)],
                                                m_v.at[b], sm[b])
            ha = {}
            for b in (0, 1):
                hi[("i", b)].wait()
                hi[("m", b)].wait()
                ha[b] = pltpu.async_copy(m_v.at[b],
                                         agg_sh.at[idx_v.at[b].at[0]],
                                         sa[b], add=True)
            for b in (0, 1):
                ha[b].wait()

        @pl.loop(pairs_end, cnt_w)
        def _(i):
            j = wid + i * NW
            pltpu.sync_copy(rows_hbm.at[j], idx_v.at[0])
            pltpu.sync_copy(m_hbm.at[pl.ds(j * CH, CH)], m_v.at[0])
            pltpu.sync_copy(m_v.at[0], agg_sh.at[idx_v.at[0].at[0]], add=True)

        plsc.subcore_barrier()

        @pl.loop(0, RP // CH)
        def _(k):
            pltpu.sync_copy(agg_sh.at[pl.ds(base + k * CH, CH)], m_v.at[0])
            pltpu.sync_copy(m_v.at[0],
                            p_hbm.at[pl.ds(cid * NP + base + k * CH, CH)])

        if not with_counts:
            return

        pltpu.sync_copy(z128_hbm, m_v.at[0])
        pltpu.sync_copy(ones_hbm, m_v.at[1])
        plsc.subcore_barrier()

        @pl.loop(0, RP // CH)
        def _(k):
            pltpu.sync_copy(m_v.at[0], agg_sh.at[pl.ds(base + k * CH, CH)])

        plsc.subcore_barrier()
        cntall_w = (NCHUNK - wid + NW - 1) // NW
        pairs_all = (cntall_w // 2) * 2

        @pl.loop(0, pairs_all, step=2)
        def _(kk):
            hi = {}
            for b in (0, 1):
                j = wid + (kk + b) * NW
                hi[b] = pltpu.async_copy(rows_all_hbm.at[j], idx_v.at[b],
                                         si[b])
            ha = {}
            for b in (0, 1):
                hi[b].wait()
                ha[b] = pltpu.async_copy(m_v.at[1],
                                         acc_at_idx(agg_sh, idx_v, b),
                                         sa[b], add=True)
            for b in (0, 1):
                ha[b].wait()

        @pl.loop(pairs_all, cntall_w)
        def _(i):
            j = wid + i * NW
            pltpu.sync_copy(rows_all_hbm.at[j], idx_v.at[0])
            pltpu.sync_copy(m_v.at[1], agg_sh.at[idx_v.at[0].at[0]], add=True)

        plsc.subcore_barrier()

        @pl.loop(0, RP // CH)
        def _(k):
            pltpu.sync_copy(agg_sh.at[pl.ds(base + k * CH, CH)], m_v.at[0])
            pltpu.sync_copy(m_v.at[0],
                            c_hbm.at[pl.ds(cid * NP + base + k * CH, CH)])

    scratch = [
        pltpu.VMEM_SHARED((NP, D), jnp.float32),
        pltpu.VMEM((2, 1, CH), jnp.int32),
        pltpu.VMEM((2, CH, D), jnp.float32),
    ] + [pltpu.SemaphoreType.DMA] * 6
    return functools.partial(pl.kernel, out_type=tuple(outs) if with_counts
                             else outs[0], mesh=_MESH,
                             scratch_types=scratch)(scatter)


EB = 2000


def _tc_mlp_body(xs_ref, xt_ref, ea_ref, w1bt_ref, w1a_ref, b1_ref, w2_ref,
                 m_ref):
    d = xs_ref[...] - xt_ref[...]
    radial = jnp.sqrt(jnp.sum(d * d, axis=1, keepdims=True))
    z = jnp.dot(ea_ref[...].astype(jnp.bfloat16), w1bt_ref[...],
                preferred_element_type=jnp.float32).astype(jnp.bfloat16)
    z = z + radial.astype(jnp.bfloat16) * w1a_ref[...] + b1_ref[...]
    h = jnp.maximum(z, jnp.bfloat16(0.2) * z)
    e = jnp.sum(h * w2_ref[...], axis=1, keepdims=True)
    m_ref[...] = d * jnp.tanh(e.astype(jnp.float32))


def _tc_mlp(xs, xt, ea, w1bt, w1a, b1, w2):
    e_sl = xs.shape[0]
    return pl.pallas_call(
        _tc_mlp_body,
        grid=(e_sl // EB,),
        in_specs=[
            pl.BlockSpec((EB, D), lambda i: (i, 0)),
            pl.BlockSpec((EB, D), lambda i: (i, 0)),
            pl.BlockSpec((EB, EA), lambda i: (i, 0)),
            pl.BlockSpec((EA, NF), lambda i: (0, 0)),
            pl.BlockSpec((1, NF), lambda i: (0, 0)),
            pl.BlockSpec((1, NF), lambda i: (0, 0)),
            pl.BlockSpec((1, NF), lambda i: (0, 0)),
        ],
        out_specs=pl.BlockSpec((EB, D), lambda i: (i, 0)),
        out_shape=jax.ShapeDtypeStruct((e_sl, D), jnp.float32),
    )(xs, xt, ea, w1bt, w1a, b1, w2)


NB = 2000


def _tc_combine_body(x_ref, p_ref, c_ref, vel_ref, vn_ref,
                     wv1_ref, bv1_ref, wv2_ref, bv2_ref, o_ref):
    cnt = c_ref[0, :, 0:1] + c_ref[1, :, 0:1]
    agg = p_ref[0]
    for q in range(1, K * NC):
        agg = agg + p_ref[q]
    agg = agg / jnp.maximum(cnt, 1.0)
    zv = vn_ref[...] * wv1_ref[...] + bv1_ref[...]
    hv = jnp.maximum(zv, 0.2 * zv)
    v_scale = jnp.sum(hv * wv2_ref[...], axis=1, keepdims=True) + bv2_ref[...]
    o_ref[...] = x_ref[...] + agg + vel_ref[...] * v_scale


def _tc_combine(x, p, c, vel, vn, wv1, bv1, wv2, bv2):
    return pl.pallas_call(
        _tc_combine_body,
        grid=(N // NB,),
        in_specs=[
            pl.BlockSpec((NB, D), lambda i: (i, 0)),
            pl.BlockSpec((K * NC, NB, D), lambda i: (0, i, 0)),
            pl.BlockSpec((NC, NB, D), lambda i: (0, i, 0)),
            pl.BlockSpec((NB, D), lambda i: (i, 0)),
            pl.BlockSpec((NB, 1), lambda i: (i, 0)),
            pl.BlockSpec((1, NF), lambda i: (0, 0)),
            pl.BlockSpec((1, NF), lambda i: (0, 0)),
            pl.BlockSpec((1, NF), lambda i: (0, 0)),
            pl.BlockSpec((1, 1), lambda i: (0, 0)),
        ],
        out_specs=pl.BlockSpec((NB, D), lambda i: (i, 0)),
        out_shape=jax.ShapeDtypeStruct((N, D), jnp.float32),
    )(x, p, c, vel, vn, wv1, bv1, wv2, bv2)


def kernel(x, vel_norm, vel, edge_index, edge_attr, W1, b1, W2,
           Wv1, bv1, Wv2, bv2):
    row = edge_index[0].astype(jnp.int32)
    col = edge_index[1].astype(jnp.int32)
    rows3d = row.reshape(NCHUNK, 1, CH)
    cols3d = col.reshape(NCHUNK, 1, CH)

    w1bt = W1[:, 1:].T.astype(jnp.bfloat16)
    w1a = W1[:, 0].reshape(1, NF).astype(jnp.bfloat16)
    b1r = b1.reshape(1, NF).astype(jnp.bfloat16)
    w2r = W2.reshape(1, NF).astype(jnp.bfloat16)

    z128 = jnp.zeros((CH, D), jnp.float32)
    ones128 = jnp.ones((CH, D), jnp.float32)

    gather = _make_gather(NCS)
    parts = []
    c = None
    for k in range(K):
        r_sl = rows3d[k * NCS:(k + 1) * NCS]
        c_sl = cols3d[k * NCS:(k + 1) * NCS]
        ea_sl = edge_attr[k * ES:(k + 1) * ES]
        xs, xt = gather(x, r_sl, c_sl)
        m = _tc_mlp(xs, xt, ea_sl, w1bt, w1a, b1r, w2r)
        if k == 0:
            pk, c = _make_scatter(NCS, True)(m, rows3d, r_sl, z128, ones128)
        else:
            pk = _make_scatter(NCS)(m, rows3d, r_sl, z128, ones128)
        parts.append(pk)

    p = jnp.concatenate(parts).reshape(K * NC, NP, D)
    c = c.reshape(NC, NP, D)

    wv1 = Wv1.reshape(1, NF)
    bv1r = bv1.reshape(1, NF)
    wv2 = Wv2.reshape(1, NF)
    bv2r = bv2.reshape(1, 1)
    x_out = _tc_combine(x, p, c, vel, vel_norm, wv1, bv1r, wv2, bv2r)
    return (x_out, edge_attr)


# speedup vs baseline: 1.0276x; 1.0009x over previous
import functools

import jax
import jax.numpy as jnp
from jax import lax
from jax.experimental import pallas as pl
from jax.experimental.pallas import tpu as pltpu
from jax.experimental.pallas import tpu_sc as plsc

N = 10000
E = 320000
D = 128
NF = 1024
EA = 16

NC = 2
NS = 16
NW = NC * NS
CH = 128
NCHUNK = E // CH
NP = 10240
RP = NP // NS
K = 2
ES = E // K
NCS = NCHUNK // K

def acc_at_idx(acc, idx_v, b):
    return acc.at[idx_v.at[b].at[0]]


_MESH = plsc.VectorSubcoreMesh(core_axis_name="c", subcore_axis_name="s",
                               num_cores=NC, num_subcores=NS)


@functools.cache
def _make_gather(nchunk):
    e_sl = nchunk * CH

    @functools.partial(
        pl.kernel,
        out_type=(jax.ShapeDtypeStruct((e_sl, D), jnp.float32),
                  jax.ShapeDtypeStruct((e_sl, D), jnp.float32)),
        mesh=_MESH,
        scratch_types=[
            pltpu.VMEM((2, 1, CH), jnp.int32),
            pltpu.VMEM((2, 1, CH), jnp.int32),
            pltpu.VMEM((2, CH, D), jnp.float32),
            pltpu.VMEM((2, CH, D), jnp.float32),
        ] + [pltpu.SemaphoreType.DMA] * 12,
        compiler_params=pltpu.CompilerParams(use_tc_tiling_on_sc=True),
    )
    def gather(x_hbm, rows_hbm, cols_hbm, xs_hbm, xt_hbm,
               idxr_v, idxc_v, xs_v, xt_v,
               sir0, sir1, sic0, sic1, sgr0, sgr1, sgc0, sgc1,
               swr0, swr1, swc0, swc1):
        wid = lax.axis_index("s") * NC + lax.axis_index("c")
        sir = (sir0, sir1); sic = (sic0, sic1)
        sgr = (sgr0, sgr1); sgc = (sgc0, sgc1)
        swr = (swr0, swr1); swc = (swc0, swc1)
        cnt_w = (nchunk - wid + NW - 1) // NW
        pairs_end = (cnt_w // 2) * 2

        @pl.loop(0, pairs_end, step=2)
        def _(kk):
            hi = {}
            for b in (0, 1):
                j = wid + (kk + b) * NW
                hi[("r", b)] = pltpu.async_copy(rows_hbm.at[j], idxr_v.at[b],
                                                sir[b])
                hi[("c", b)] = pltpu.async_copy(cols_hbm.at[j], idxc_v.at[b],
                                                sic[b])
            hg = {}
            for b in (0, 1):
                hi[("r", b)].wait()
                hg[("r", b)] = pltpu.async_copy(x_hbm.at[idxr_v.at[b].at[0]],
                                                xs_v.at[b], sgr[b])
                hi[("c", b)].wait()
                hg[("c", b)] = pltpu.async_copy(x_hbm.at[idxc_v.at[b].at[0]],
                                                xt_v.at[b], sgc[b])
            hw = {}
            for b in (0, 1):
                j = wid + (kk + b) * NW
                hg[("r", b)].wait()
                hw[("r", b)] = pltpu.async_copy(
                    xs_v.at[b], xs_hbm.at[pl.ds(j * CH, CH)], swr[b])
                hg[("c", b)].wait()
                hw[("c", b)] = pltpu.async_copy(
                    xt_v.at[b], xt_hbm.at[pl.ds(j * CH, CH)], swc[b])
            for b in (0, 1):
                hw[("r", b)].wait()
                hw[("c", b)].wait()

        @pl.loop(pairs_end, cnt_w)
        def _(i):
            j = wid + i * NW
            pltpu.sync_copy(rows_hbm.at[j], idxr_v.at[0])
            pltpu.sync_copy(cols_hbm.at[j], idxc_v.at[0])
            pltpu.sync_copy(x_hbm.at[idxr_v.at[0].at[0]], xs_v.at[0])
            pltpu.sync_copy(x_hbm.at[idxc_v.at[0].at[0]], xt_v.at[0])
            pltpu.sync_copy(xs_v.at[0], xs_hbm.at[pl.ds(j * CH, CH)])
            pltpu.sync_copy(xt_v.at[0], xt_hbm.at[pl.ds(j * CH, CH)])

    return gather


@functools.cache
def _make_scatter(nchunk, with_counts=False):
    outs = [jax.ShapeDtypeStruct((NC * NP, D), jnp.float32)]
    if with_counts:
        outs.append(jax.ShapeDtypeStruct((NC * NP, D), jnp.float32))

    def scatter(m_hbm, rows_all_hbm, rows_hbm, z128_hbm, ones_hbm,
                p_hbm, *rest):
        if with_counts:
            c_hbm, agg_sh, idx_v, m_v, si0, si1, sm0, sm1, sa0, sa1 = rest
        else:
            agg_sh, idx_v, m_v, si0, si1, sm0, sm1, sa0, sa1 = rest
        cid = lax.axis_index("c")
        sid = lax.axis_index("s")
        wid = sid * NC + cid
        base = sid * RP
        si = (si0, si1); sm = (sm0, sm1); sa = (sa0, sa1)
        cnt_w = (nchunk - wid + NW - 1) // NW
        pairs_end = (cnt_w // 2) * 2

        pltpu.sync_copy(z128_hbm, m_v.at[0])

        @pl.loop(0, RP // CH)
        def _(k):
            pltpu.sync_copy(m_v.at[0], agg_sh.at[pl.ds(base + k * CH, CH)])

        plsc.subcore_barrier()

        @pl.loop(0, pairs_end, step=2)
        def _(kk):
            hi = {}
            for b in (0, 1):
                j = wid + (kk + b) * NW
                hi[("i", b)] = pltpu.async_copy(rows_hbm.at[j], idx_v.at[b],
                                                si[b])
                hi[("m", b)] = pltpu.async_copy(m_hbm.at[pl.ds(j * CH, CH)],
                                                m_v.at[b], sm[b])
            ha = {}
            for b in (0, 1):
                hi[("i", b)].wait()
                hi[("m", b)].wait()
                ha[b] = pltpu.async_copy(m_v.at[b],
                                         agg_sh.at[idx_v.at[b].at[0]],
                                         sa[b], add=True)
            for b in (0, 1):
                ha[b].wait()

        @pl.loop(pairs_end, cnt_w)
        def _(i):
            j = wid + i * NW
            pltpu.sync_copy(rows_hbm.at[j], idx_v.at[0])
            pltpu.sync_copy(m_hbm.at[pl.ds(j * CH, CH)], m_v.at[0])
            pltpu.sync_copy(m_v.at[0], agg_sh.at[idx_v.at[0].at[0]], add=True)

        plsc.subcore_barrier()

        @pl.loop(0, RP // CH)
        def _(k):
            pltpu.sync_copy(agg_sh.at[pl.ds(base + k * CH, CH)], m_v.at[0])
            pltpu.sync_copy(m_v.at[0],
                            p_hbm.at[pl.ds(cid * NP + base + k * CH, CH)])

        if not with_counts:
            return

        pltpu.sync_copy(z128_hbm, m_v.at[0])
        pltpu.sync_copy(ones_hbm, m_v.at[1])
        plsc.subcore_barrier()

        @pl.loop(0, RP // CH)
        def _(k):
            pltpu.sync_copy(m_v.at[0], agg_sh.at[pl.ds(base + k * CH, CH)])

        plsc.subcore_barrier()
        cntall_w = (NCHUNK - wid + NW - 1) // NW
        pairs_all = (cntall_w // 2) * 2

        @pl.loop(0, pairs_all, step=2)
        def _(kk):
            hi = {}
            for b in (0, 1):
                j = wid + (kk + b) * NW
                hi[b] = pltpu.async_copy(rows_all_hbm.at[j], idx_v.at[b],
                                         si[b])
            ha = {}
            for b in (0, 1):
                hi[b].wait()
                ha[b] = pltpu.async_copy(m_v.at[1],
                                         acc_at_idx(agg_sh, idx_v, b),
                                         sa[b], add=True)
            for b in (0, 1):
                ha[b].wait()

        @pl.loop(pairs_all, cntall_w)
        def _(i):
            j = wid + i * NW
            pltpu.sync_copy(rows_all_hbm.at[j], idx_v.at[0])
            pltpu.sync_copy(m_v.at[1], agg_sh.at[idx_v.at[0].at[0]], add=True)

        plsc.subcore_barrier()

        @pl.loop(0, RP // CH)
        def _(k):
            pltpu.sync_copy(agg_sh.at[pl.ds(base + k * CH, CH)], m_v.at[0])
            pltpu.sync_copy(m_v.at[0],
                            c_hbm.at[pl.ds(cid * NP + base + k * CH, CH)])

    scratch = [
        pltpu.VMEM_SHARED((NP, D), jnp.float32),
        pltpu.VMEM((2, 1, CH), jnp.int32),
        pltpu.VMEM((2, CH, D), jnp.float32),
    ] + [pltpu.SemaphoreType.DMA] * 6
    cp = pltpu.CompilerParams(use_tc_tiling_on_sc=True)
    return functools.partial(pl.kernel, out_type=tuple(outs) if with_counts
                             else outs[0], mesh=_MESH, compiler_params=cp,
                             scratch_types=scratch)(scatter)


EB = 2000


def _tc_mlp_body(xs_ref, xt_ref, ea_ref, w1bt_ref, w1a_ref, b1_ref, w2_ref,
                 m_ref):
    d = xs_ref[...] - xt_ref[...]
    radial = jnp.sqrt(jnp.sum(d * d, axis=1, keepdims=True))
    z = jnp.dot(ea_ref[...].astype(jnp.bfloat16), w1bt_ref[...],
                preferred_element_type=jnp.float32).astype(jnp.bfloat16)
    z = z + radial.astype(jnp.bfloat16) * w1a_ref[...] + b1_ref[...]
    h = jnp.maximum(z, jnp.bfloat16(0.2) * z)
    e = jnp.sum(h * w2_ref[...], axis=1, keepdims=True)
    m_ref[...] = d * jnp.tanh(e.astype(jnp.float32))


def _tc_mlp(xs, xt, ea, w1bt, w1a, b1, w2):
    e_sl = xs.shape[0]
    return pl.pallas_call(
        _tc_mlp_body,
        grid=(e_sl // EB,),
        in_specs=[
            pl.BlockSpec((EB, D), lambda i: (i, 0)),
            pl.BlockSpec((EB, D), lambda i: (i, 0)),
            pl.BlockSpec((EB, EA), lambda i: (i, 0)),
            pl.BlockSpec((EA, NF), lambda i: (0, 0)),
            pl.BlockSpec((1, NF), lambda i: (0, 0)),
            pl.BlockSpec((1, NF), lambda i: (0, 0)),
            pl.BlockSpec((1, NF), lambda i: (0, 0)),
        ],
        out_specs=pl.BlockSpec((EB, D), lambda i: (i, 0)),
        out_shape=jax.ShapeDtypeStruct((e_sl, D), jnp.float32),
    )(xs, xt, ea, w1bt, w1a, b1, w2)


NB = 2000


def _tc_combine_body(x_ref, p_ref, c_ref, vel_ref, vn_ref,
                     wv1_ref, bv1_ref, wv2_ref, bv2_ref, o_ref):
    cnt = c_ref[0, :, 0:1] + c_ref[1, :, 0:1]
    agg = p_ref[0]
    for q in range(1, K * NC):
        agg = agg + p_ref[q]
    agg = agg / jnp.maximum(cnt, 1.0)
    zv = vn_ref[...] * wv1_ref[...] + bv1_ref[...]
    hv = jnp.maximum(zv, 0.2 * zv)
    v_scale = jnp.sum(hv * wv2_ref[...], axis=1, keepdims=True) + bv2_ref[...]
    o_ref[...] = x_ref[...] + agg + vel_ref[...] * v_scale


def _tc_combine(x, p, c, vel, vn, wv1, bv1, wv2, bv2):
    return pl.pallas_call(
        _tc_combine_body,
        grid=(N // NB,),
        in_specs=[
            pl.BlockSpec((NB, D), lambda i: (i, 0)),
            pl.BlockSpec((K * NC, NB, D), lambda i: (0, i, 0)),
            pl.BlockSpec((NC, NB, D), lambda i: (0, i, 0)),
            pl.BlockSpec((NB, D), lambda i: (i, 0)),
            pl.BlockSpec((NB, 1), lambda i: (i, 0)),
            pl.BlockSpec((1, NF), lambda i: (0, 0)),
            pl.BlockSpec((1, NF), lambda i: (0, 0)),
            pl.BlockSpec((1, NF), lambda i: (0, 0)),
            pl.BlockSpec((1, 1), lambda i: (0, 0)),
        ],
        out_specs=pl.BlockSpec((NB, D), lambda i: (i, 0)),
        out_shape=jax.ShapeDtypeStruct((N, D), jnp.float32),
    )(x, p, c, vel, vn, wv1, bv1, wv2, bv2)


def kernel(x, vel_norm, vel, edge_index, edge_attr, W1, b1, W2,
           Wv1, bv1, Wv2, bv2):
    row = edge_index[0].astype(jnp.int32)
    col = edge_index[1].astype(jnp.int32)
    rows3d = row.reshape(NCHUNK, 1, CH)
    cols3d = col.reshape(NCHUNK, 1, CH)

    w1bt = W1[:, 1:].T.astype(jnp.bfloat16)
    w1a = W1[:, 0].reshape(1, NF).astype(jnp.bfloat16)
    b1r = b1.reshape(1, NF).astype(jnp.bfloat16)
    w2r = W2.reshape(1, NF).astype(jnp.bfloat16)

    z128 = jnp.zeros((CH, D), jnp.float32)
    ones128 = jnp.ones((CH, D), jnp.float32)

    gather = _make_gather(NCS)
    parts = []
    c = None
    for k in range(K):
        r_sl = rows3d[k * NCS:(k + 1) * NCS]
        c_sl = cols3d[k * NCS:(k + 1) * NCS]
        ea_sl = edge_attr[k * ES:(k + 1) * ES]
        xs, xt = gather(x, r_sl, c_sl)
        m = _tc_mlp(xs, xt, ea_sl, w1bt, w1a, b1r, w2r)
        if k == 0:
            pk, c = _make_scatter(NCS, True)(m, rows3d, r_sl, z128, ones128)
        else:
            pk = _make_scatter(NCS)(m, rows3d, r_sl, z128, ones128)
        parts.append(pk)

    p = jnp.concatenate(parts).reshape(K * NC, NP, D)
    c = c.reshape(NC, NP, D)

    wv1 = Wv1.reshape(1, NF)
    bv1r = bv1.reshape(1, NF)
    wv2 = Wv2.reshape(1, NF)
    bv2r = bv2.reshape(1, 1)
    x_out = _tc_combine(x, p, c, vel, vel_norm, wv1, bv1r, wv2, bv2r)
    return (x_out, edge_attr)
